# baseline (speedup 1.0000x reference)
import functools

import jax
import jax.numpy as jnp
from jax import lax
from jax.experimental import pallas as pl
from jax.experimental.pallas import tpu as pltpu
from jax.experimental.pallas import tpu_sc as plsc

_B = 1024
_BW = _B // 4
_NB = 12
_L = 16
_NW = 32
_G = 8


def _mem_words(mem):
    n = mem.shape[0]
    return lax.bitcast_convert_type(
        mem.reshape(n, 1024, 4).astype(jnp.uint8), jnp.int32)


def _ram_layer_sc(bitsT, conn, memw, out_rows=None, out_offset=0,
                  zero_tail=False, passthrough_head=False):
    N = conn.shape[0]
    conn_flat = conn.reshape(N * _NB)
    npw = N // _NW
    nch = npw // _G
    if out_rows is None:
        out_rows = N
    ztail = out_rows - N if zero_tail else 0
    zpw = ztail // _NW
    mesh = plsc.VectorSubcoreMesh(core_axis_name="c", subcore_axis_name="s")

    @functools.partial(
        pl.kernel,
        out_type=jax.ShapeDtypeStruct((out_rows, _BW), jnp.int32),
        mesh=mesh,
        scratch_types=[
            pltpu.VMEM((npw * _NB,), jnp.int32),
            pltpu.VMEM((2, _G * _NB, _BW), jnp.int32),
            pltpu.VMEM((npw, 1024), jnp.int32),
            pltpu.VMEM((2 * _G, _BW), jnp.int32),
            pltpu.SemaphoreType.DMA,
            pltpu.SemaphoreType.DMA,
            pltpu.SemaphoreType.DMA,
        ],
        compiler_params=pltpu.CompilerParams(needs_layout_passes=False),
    )
    def layer(bitsT_hbm, conn_hbm, memw_hbm, out_hbm,
              conn_v, cols_v, memc_v, out_v, cs0, cs1, msem):
        csem = (cs0, cs1)
        wid = lax.axis_index("s") * 2 + lax.axis_index("c")
        base = wid * npw
        pltpu.sync_copy(conn_hbm.at[pl.ds(base * _NB, npw * _NB)], conn_v)

        def issue(c, b):
            idx = conn_v.at[pl.ds(c * (_G * _NB), _G * _NB)]
            pltpu.async_copy(bitsT_hbm.at[idx], cols_v.at[b], csem[b])

        issue(0, 0)
        mem_cp = pltpu.async_copy(memw_hbm.at[pl.ds(base, npw)], memc_v,
                                  msem)

        if passthrough_head:
            hpw = out_offset // _NW
            for r in range(hpw // (2 * _G)):
                src = pl.ds(wid * hpw + r * 2 * _G, 2 * _G)
                pltpu.sync_copy(bitsT_hbm.at[src], out_v)
                pltpu.sync_copy(out_v, out_hbm.at[src])

        if ztail:
            z = jnp.zeros((_L,), jnp.int32)
            for j in range(2 * _G):
                for t in range(_BW // _L):
                    out_v[j, pl.ds(t * _L, _L)] = z
            for i in range(zpw // (2 * _G)):
                pltpu.sync_copy(
                    out_v,
                    out_hbm.at[pl.ds(out_offset + N + wid * zpw
                                     + i * 2 * _G, 2 * _G)])

        mem_cp.wait()

        def body(g, carry):
            for b in (0, 1):
                c = 2 * g + b
                issue(jnp.minimum(c + 1, nch - 1), 1 - b)
                pltpu.make_async_copy(
                    bitsT_hbm.at[conn_v.at[pl.ds(0, _G * _NB)]],
                    cols_v.at[b], csem[b]).wait()

                def group(t, carry2):
                    sl = pl.ds(t * _L, _L)
                    for j in range(_G):
                        lo = cols_v[b, j * _NB, sl]
                        for k in range(1, 6):
                            lo = lo | (cols_v[b, j * _NB + k, sl] << k)
                        hi = cols_v[b, j * _NB + 6, sl]
                        for k in range(7, _NB):
                            hi = hi | (cols_v[b, j * _NB + k, sl] << (k - 6))
                        row = jnp.full((_L,), (2 * g + b) * _G + j,
                                       jnp.int32)
                        packed = None
                        for q in range(4):
                            addr = ((lo >> (8 * q)) & 63) | \
                                   (((hi >> (8 * q)) & 63) << 6)
                            word = plsc.load_gather(memc_v, [row, addr >> 2])
                            bit = (word >> ((addr & 3) << 3)) & 1
                            bit = bit << (8 * q)
                            packed = bit if packed is None else packed | bit
                        out_v[b * _G + j, sl] = packed
                    return carry2

                lax.fori_loop(0, _BW // _L, group, 0)
            pltpu.sync_copy(
                out_v,
                out_hbm.at[pl.ds(out_offset + base + g * 2 * _G, 2 * _G)])
            return carry

        lax.fori_loop(0, nch // 2, body, 0)
        pltpu.make_async_copy(
            bitsT_hbm.at[conn_v.at[pl.ds(0, _G * _NB)]],
            cols_v.at[0], csem[0]).wait()

    return layer(bitsT, conn_flat, memw)


def _to_words(bitsT_u8):
    t = bitsT_u8.shape[0]
    return lax.bitcast_convert_type(bitsT_u8.reshape(t, _BW, 4), jnp.int32)


def kernel(input, conn_in, conn_state, conn_out, mem_in, mem_state, mem_out):
    bitsT = _to_words(input.T.astype(jnp.uint8))
    out1T = _ram_layer_sc(bitsT, conn_in, _mem_words(mem_in),
                          out_rows=4096, zero_tail=True)
    out2T = _ram_layer_sc(out1T, conn_state, _mem_words(mem_state),
                          out_rows=4096, out_offset=2048,
                          passthrough_head=True)
    outT = _ram_layer_sc(out2T, conn_out, _mem_words(mem_out))
    out_u8 = lax.bitcast_convert_type(outT, jnp.uint8).reshape(1024, _B)
    return out_u8.T.astype(jnp.bool_)


# speedup vs baseline: 1.1883x; 1.1883x over previous
import functools

import jax
import jax.numpy as jnp
from jax import lax
from jax.experimental import pallas as pl
from jax.experimental.pallas import tpu as pltpu
from jax.experimental.pallas import tpu_sc as plsc

_B = 1024
_BW = _B // 4
_NB = 12
_L = 16
_NW = 32
_G = 8


def _pack_mem(mem):
    n = mem.shape[0]
    j = jnp.arange(32)
    lo_w = jnp.where(j < 16, 2.0 ** j, 0.0).astype(jnp.float32)
    hi_w = jnp.where(j >= 16, 2.0 ** (j - 16), 0.0).astype(jnp.float32)
    wm = jnp.stack([lo_w, hi_w], axis=1)
    lohi = mem.reshape(n * 128, 32).astype(jnp.float32) @ wm
    lohi = lohi.astype(jnp.int32)
    packed = lohi[:, 0] | (lohi[:, 1] << 16)
    return packed.reshape(n, 128)


def _ram_layer_sc(bitsT, conn, memw, out_rows=None, out_offset=0,
                  zero_tail=False, passthrough_head=False):
    N = conn.shape[0]
    conn_flat = conn.reshape(N * _NB)
    npw = N // _NW
    nch = npw // _G
    if out_rows is None:
        out_rows = N
    ztail = out_rows - N if zero_tail else 0
    zpw = ztail // _NW
    mesh = plsc.VectorSubcoreMesh(core_axis_name="c", subcore_axis_name="s")

    @functools.partial(
        pl.kernel,
        out_type=jax.ShapeDtypeStruct((out_rows, _BW), jnp.int32),
        mesh=mesh,
        scratch_types=[
            pltpu.VMEM((npw * _NB,), jnp.int32),
            pltpu.VMEM((2, _G * _NB, _BW), jnp.int32),
            pltpu.VMEM((2 * _G, 128), jnp.int32),
            pltpu.VMEM((2 * _G, _BW), jnp.int32),
            pltpu.SemaphoreType.DMA,
            pltpu.SemaphoreType.DMA,
        ],
        compiler_params=pltpu.CompilerParams(needs_layout_passes=False),
    )
    def layer(bitsT_hbm, conn_hbm, memw_hbm, out_hbm,
              conn_v, cols_v, memc_v, out_v, cs0, cs1):
        csem = (cs0, cs1)
        wid = lax.axis_index("s") * 2 + lax.axis_index("c")
        base = wid * npw
        pltpu.sync_copy(conn_hbm.at[pl.ds(base * _NB, npw * _NB)], conn_v)

        def issue(c, b):
            idx = conn_v.at[pl.ds(c * (_G * _NB), _G * _NB)]
            pltpu.async_copy(bitsT_hbm.at[idx], cols_v.at[b], csem[b])

        issue(0, 0)

        if passthrough_head:
            hpw = out_offset // _NW
            for r in range(hpw // (2 * _G)):
                src = pl.ds(wid * hpw + r * 2 * _G, 2 * _G)
                pltpu.sync_copy(bitsT_hbm.at[src], out_v)
                pltpu.sync_copy(out_v, out_hbm.at[src])

        if ztail:
            z = jnp.zeros((_L,), jnp.int32)
            for j in range(2 * _G):
                for t in range(_BW // _L):
                    out_v[j, pl.ds(t * _L, _L)] = z
            for i in range(zpw // (2 * _G)):
                pltpu.sync_copy(
                    out_v,
                    out_hbm.at[pl.ds(out_offset + N + wid * zpw
                                     + i * 2 * _G, 2 * _G)])

        def body(g, carry):
            pltpu.sync_copy(memw_hbm.at[pl.ds(base + g * 2 * _G, 2 * _G)],
                            memc_v)
            for b in (0, 1):
                c = 2 * g + b
                issue(jnp.minimum(c + 1, nch - 1), 1 - b)
                pltpu.make_async_copy(
                    bitsT_hbm.at[conn_v.at[pl.ds(0, _G * _NB)]],
                    cols_v.at[b], csem[b]).wait()

                def group(t, carry2):
                    sl = pl.ds(t * _L, _L)
                    for j in range(_G):
                        lo = cols_v[b, j * _NB, sl]
                        for k in range(1, 6):
                            lo = lo | (cols_v[b, j * _NB + k, sl] << k)
                        hi = cols_v[b, j * _NB + 6, sl]
                        for k in range(7, _NB):
                            hi = hi | (cols_v[b, j * _NB + k, sl] << (k - 6))
                        row = jnp.full((_L,), b * _G + j, jnp.int32)
                        packed = None
                        for q in range(4):
                            addr = ((lo >> (8 * q)) & 63) | \
                                   (((hi >> (8 * q)) & 63) << 6)
                            word = plsc.load_gather(memc_v, [row, addr >> 5])
                            bit = (word >> (addr & 31)) & 1
                            bit = bit << (8 * q)
                            packed = bit if packed is None else packed | bit
                        out_v[b * _G + j, sl] = packed
                    return carry2

                lax.fori_loop(0, _BW // _L, group, 0)
            pltpu.sync_copy(
                out_v,
                out_hbm.at[pl.ds(out_offset + base + g * 2 * _G, 2 * _G)])
            return carry

        lax.fori_loop(0, nch // 2, body, 0)
        pltpu.make_async_copy(
            bitsT_hbm.at[conn_v.at[pl.ds(0, _G * _NB)]],
            cols_v.at[0], csem[0]).wait()

    return layer(bitsT, conn_flat, memw)


def _to_words(bitsT_u8):
    t = bitsT_u8.shape[0]
    return lax.bitcast_convert_type(bitsT_u8.reshape(t, _BW, 4), jnp.int32)


def kernel(input, conn_in, conn_state, conn_out, mem_in, mem_state, mem_out):
    bitsT = _to_words(input.T.astype(jnp.uint8))
    out1T = _ram_layer_sc(bitsT, conn_in, _pack_mem(mem_in),
                          out_rows=4096, zero_tail=True)
    out2T = _ram_layer_sc(out1T, conn_state, _pack_mem(mem_state),
                          out_rows=4096, out_offset=2048,
                          passthrough_head=True)
    outT = _ram_layer_sc(out2T, conn_out, _pack_mem(mem_out))
    out_u8 = lax.bitcast_convert_type(outT, jnp.uint8).reshape(1024, _B)
    return out_u8.T.astype(jnp.bool_)


# speedup vs baseline: 1.5290x; 1.2866x over previous
import functools

import jax
import jax.numpy as jnp
from jax import lax
from jax.experimental import pallas as pl
from jax.experimental.pallas import tpu as pltpu
from jax.experimental.pallas import tpu_sc as plsc

_B = 1024
_BW = _B // 4
_NB = 12
_L = 16
_NW = 32
_G = 8


def _pack_mem(mem):
    n = mem.shape[0]
    w = mem.astype(jnp.uint32).reshape(n, 32, 128)
    w = w << jnp.arange(32, dtype=jnp.uint32)[None, :, None]
    return lax.bitcast_convert_type(w.sum(axis=1), jnp.int32)


def _ram_layer_sc(bitsT, conn, memw, out_rows=None, out_offset=0,
                  zero_tail=False, passthrough_head=False):
    N = conn.shape[0]
    conn_flat = conn.reshape(N * _NB)
    npw = N // _NW
    nch = npw // _G
    if out_rows is None:
        out_rows = N
    ztail = out_rows - N if zero_tail else 0
    zpw = ztail // _NW
    mesh = plsc.VectorSubcoreMesh(core_axis_name="c", subcore_axis_name="s")

    @functools.partial(
        pl.kernel,
        out_type=jax.ShapeDtypeStruct((out_rows, _BW), jnp.int32),
        mesh=mesh,
        scratch_types=[
            pltpu.VMEM((npw * _NB,), jnp.int32),
            pltpu.VMEM((2, _G * _NB, _BW), jnp.int32),
            pltpu.VMEM((2 * _G, 128), jnp.int32),
            pltpu.VMEM((2 * _G, _BW), jnp.int32),
            pltpu.SemaphoreType.DMA,
            pltpu.SemaphoreType.DMA,
        ],
        compiler_params=pltpu.CompilerParams(needs_layout_passes=False),
    )
    def layer(bitsT_hbm, conn_hbm, memw_hbm, out_hbm,
              conn_v, cols_v, memc_v, out_v, cs0, cs1):
        csem = (cs0, cs1)
        wid = lax.axis_index("s") * 2 + lax.axis_index("c")
        base = wid * npw
        pltpu.sync_copy(conn_hbm.at[pl.ds(base * _NB, npw * _NB)], conn_v)

        def issue(c, b):
            idx = conn_v.at[pl.ds(c * (_G * _NB), _G * _NB)]
            pltpu.async_copy(bitsT_hbm.at[idx], cols_v.at[b], csem[b])

        issue(0, 0)

        if passthrough_head:
            hpw = out_offset // _NW
            for r in range(hpw // (2 * _G)):
                src = pl.ds(wid * hpw + r * 2 * _G, 2 * _G)
                pltpu.sync_copy(bitsT_hbm.at[src], out_v)
                pltpu.sync_copy(out_v, out_hbm.at[src])

        if ztail:
            z = jnp.zeros((_L,), jnp.int32)
            for j in range(2 * _G):
                for t in range(_BW // _L):
                    out_v[j, pl.ds(t * _L, _L)] = z
            for i in range(zpw // (2 * _G)):
                pltpu.sync_copy(
                    out_v,
                    out_hbm.at[pl.ds(out_offset + N + wid * zpw
                                     + i * 2 * _G, 2 * _G)])

        def body(g, carry):
            pltpu.sync_copy(memw_hbm.at[pl.ds(base + g * 2 * _G, 2 * _G)],
                            memc_v)
            for b in (0, 1):
                c = 2 * g + b
                issue(jnp.minimum(c + 1, nch - 1), 1 - b)
                pltpu.make_async_copy(
                    bitsT_hbm.at[conn_v.at[pl.ds(0, _G * _NB)]],
                    cols_v.at[b], csem[b]).wait()

                def group(t, carry2):
                    sl = pl.ds(t * _L, _L)
                    for j in range(_G):
                        lo = cols_v[b, j * _NB, sl]
                        for k in range(1, 6):
                            lo = lo | (cols_v[b, j * _NB + k, sl] << k)
                        hi = cols_v[b, j * _NB + 6, sl]
                        for k in range(7, _NB):
                            hi = hi | (cols_v[b, j * _NB + k, sl] << (k - 6))
                        row = jnp.full((_L,), b * _G + j, jnp.int32)
                        packed = None
                        for q in range(4):
                            addr = ((lo >> (8 * q)) & 63) | \
                                   (((hi >> (8 * q)) & 63) << 6)
                            word = plsc.load_gather(memc_v,
                                                    [row, addr & 127])
                            bit = (word >> (addr >> 7)) & 1
                            bit = bit << (8 * q)
                            packed = bit if packed is None else packed | bit
                        out_v[b * _G + j, sl] = packed
                    return carry2

                lax.fori_loop(0, _BW // _L, group, 0)
            pltpu.sync_copy(
                out_v,
                out_hbm.at[pl.ds(out_offset + base + g * 2 * _G, 2 * _G)])
            return carry

        lax.fori_loop(0, nch // 2, body, 0)
        pltpu.make_async_copy(
            bitsT_hbm.at[conn_v.at[pl.ds(0, _G * _NB)]],
            cols_v.at[0], csem[0]).wait()

    return layer(bitsT, conn_flat, memw)


def _to_words(bitsT_u8):
    t = bitsT_u8.shape[0]
    return lax.bitcast_convert_type(bitsT_u8.reshape(t, _BW, 4), jnp.int32)


def kernel(input, conn_in, conn_state, conn_out, mem_in, mem_state, mem_out):
    bitsT = _to_words(input.T.astype(jnp.uint8))
    out1T = _ram_layer_sc(bitsT, conn_in, _pack_mem(mem_in),
                          out_rows=4096, zero_tail=True)
    out2T = _ram_layer_sc(out1T, conn_state, _pack_mem(mem_state),
                          out_rows=4096, out_offset=2048,
                          passthrough_head=True)
    outT = _ram_layer_sc(out2T, conn_out, _pack_mem(mem_out))
    out_u8 = lax.bitcast_convert_type(outT, jnp.uint8).reshape(1024, _B)
    return out_u8.T.astype(jnp.bool_)


# speedup vs baseline: 2.2057x; 1.4426x over previous
import functools

import jax
import jax.numpy as jnp
from jax import lax
from jax.experimental import pallas as pl
from jax.experimental.pallas import tpu as pltpu
from jax.experimental.pallas import tpu_sc as plsc

_B = 1024
_BW = _B // 4
_NB = 12
_L = 16
_NW = 32
_G = 8


def _pack_mem(mem):
    n = mem.shape[0]
    w = mem.astype(jnp.uint32).reshape(n, 128, 32)
    w = w << jnp.arange(32, dtype=jnp.uint32)
    return lax.bitcast_convert_type(w.sum(axis=-1), jnp.int32)


def _ram_layer_sc(bitsT, conn, memw, out_rows=None, out_offset=0,
                  zero_tail=False, passthrough_head=False):
    N = conn.shape[0]
    conn_flat = conn.reshape(N * _NB)
    npw = N // _NW
    nch = npw // _G
    if out_rows is None:
        out_rows = N
    ztail = out_rows - N if zero_tail else 0
    zpw = ztail // _NW
    mesh = plsc.VectorSubcoreMesh(core_axis_name="c", subcore_axis_name="s")

    @functools.partial(
        pl.kernel,
        out_type=jax.ShapeDtypeStruct((out_rows, _BW), jnp.int32),
        mesh=mesh,
        scratch_types=[
            pltpu.VMEM((npw * _NB,), jnp.int32),
            pltpu.VMEM((2, _G * _NB, _BW), jnp.int32),
            pltpu.VMEM((2, 2 * _G, 128), jnp.int32),
            pltpu.VMEM((2, 2 * _G, _BW), jnp.int32),
            pltpu.SemaphoreType.DMA,
            pltpu.SemaphoreType.DMA,
            pltpu.SemaphoreType.DMA,
            pltpu.SemaphoreType.DMA,
            pltpu.SemaphoreType.DMA,
            pltpu.SemaphoreType.DMA,
        ],
        compiler_params=pltpu.CompilerParams(needs_layout_passes=False),
    )
    def layer(bitsT_hbm, conn_hbm, memw_hbm, out_hbm,
              conn_v, cols_v, memc_v, out_v, cs0, cs1, ms0, ms1, os0, os1):
        csem = (cs0, cs1)
        msem = (ms0, ms1)
        osem = (os0, os1)
        ghalf = nch // 2
        wid = lax.axis_index("s") * 2 + lax.axis_index("c")
        base = wid * npw
        pltpu.sync_copy(conn_hbm.at[pl.ds(base * _NB, npw * _NB)], conn_v)

        def issue(c, b):
            idx = conn_v.at[pl.ds(c * (_G * _NB), _G * _NB)]
            pltpu.async_copy(bitsT_hbm.at[idx], cols_v.at[b], csem[b])

        def mem_issue(g, p):
            pltpu.async_copy(memw_hbm.at[pl.ds(base + g * 2 * _G, 2 * _G)],
                             memc_v.at[p], msem[p])

        def mem_wait(p):
            pltpu.make_async_copy(memw_hbm.at[pl.ds(base, 2 * _G)],
                                  memc_v.at[p], msem[p]).wait()

        def out_issue(g, p):
            pltpu.async_copy(
                out_v.at[p],
                out_hbm.at[pl.ds(out_offset + base + g * 2 * _G, 2 * _G)],
                osem[p])

        def out_wait(p):
            pltpu.make_async_copy(
                out_v.at[p], out_hbm.at[pl.ds(out_offset + base, 2 * _G)],
                osem[p]).wait()

        issue(0, 0)
        mem_issue(0, 0)
        mem_issue(1, 1)

        if passthrough_head:
            hpw = out_offset // _NW
            for r in range(hpw // (2 * _G)):
                src = pl.ds(wid * hpw + r * 2 * _G, 2 * _G)
                pltpu.sync_copy(bitsT_hbm.at[src], out_v.at[0])
                pltpu.sync_copy(out_v.at[0], out_hbm.at[src])

        if ztail:
            z = jnp.zeros((_L,), jnp.int32)
            for j in range(2 * _G):
                for t in range(_BW // _L):
                    out_v[0, j, pl.ds(t * _L, _L)] = z
            for i in range(zpw // (2 * _G)):
                pltpu.sync_copy(
                    out_v.at[0],
                    out_hbm.at[pl.ds(out_offset + N + wid * zpw
                                     + i * 2 * _G, 2 * _G)])

        def body(h, carry):
            for p in (0, 1):
                g = 2 * h + p

                @pl.when(h > 0)
                def _():
                    out_wait(p)

                mem_wait(p)
                for b in (0, 1):
                    c = 2 * g + b
                    issue(jnp.minimum(c + 1, nch - 1), 1 - b)
                    pltpu.make_async_copy(
                        bitsT_hbm.at[conn_v.at[pl.ds(0, _G * _NB)]],
                        cols_v.at[b], csem[b]).wait()

                    def group(t, carry2):
                        sl = pl.ds(t * _L, _L)
                        for j in range(_G):
                            lo = cols_v[b, j * _NB, sl]
                            for k in range(1, 6):
                                lo = lo | (cols_v[b, j * _NB + k, sl] << k)
                            hi = cols_v[b, j * _NB + 6, sl]
                            for k in range(7, _NB):
                                hi = hi | (cols_v[b, j * _NB + k, sl]
                                           << (k - 6))
                            row = jnp.full((_L,), b * _G + j, jnp.int32)
                            packed = None
                            for q in range(4):
                                addr = ((lo >> (8 * q)) & 63) | \
                                       (((hi >> (8 * q)) & 63) << 6)
                                word = plsc.load_gather(
                                    memc_v.at[p], [row, addr >> 5])
                                bit = (word >> (addr & 31)) & 1
                                bit = bit << (8 * q)
                                packed = (bit if packed is None
                                          else packed | bit)
                            out_v[p, b * _G + j, sl] = packed
                        return carry2

                    lax.fori_loop(0, _BW // _L, group, 0)
                out_issue(g, p)
                mem_issue(jnp.minimum(g + 2, ghalf - 1), p)
            return carry

        lax.fori_loop(0, nch // 4, body, 0)
        out_wait(0)
        out_wait(1)
        mem_wait(0)
        mem_wait(1)
        pltpu.make_async_copy(
            bitsT_hbm.at[conn_v.at[pl.ds(0, _G * _NB)]],
            cols_v.at[0], csem[0]).wait()

    return layer(bitsT, conn_flat, memw)


def _to_words(bitsT_u8):
    t = bitsT_u8.shape[0]
    return lax.bitcast_convert_type(bitsT_u8.reshape(t, _BW, 4), jnp.int32)


def kernel(input, conn_in, conn_state, conn_out, mem_in, mem_state, mem_out):
    bitsT = _to_words(input.T.astype(jnp.uint8))
    out1T = _ram_layer_sc(bitsT, conn_in, _pack_mem(mem_in),
                          out_rows=4096, zero_tail=True)
    out2T = _ram_layer_sc(out1T, conn_state, _pack_mem(mem_state),
                          out_rows=4096, out_offset=2048,
                          passthrough_head=True)
    outT = _ram_layer_sc(out2T, conn_out, _pack_mem(mem_out))
    out_u8 = lax.bitcast_convert_type(outT, jnp.uint8).reshape(1024, _B)
    return out_u8.T.astype(jnp.bool_)
